# baseline (speedup 1.0000x reference)
import functools

import jax
import jax.numpy as jnp
from jax import lax
from jax.experimental import pallas as pl
from jax.experimental.pallas import tpu as pltpu
from jax.experimental.pallas import tpu_sc as plsc

VOCAB = 100000
EMBED_DIM = 128
BATCH = 4096
CTX = 20

_INFO = plsc.get_sparse_core_info()
_NC, _NS = _INFO.num_cores, _INFO.num_subcores
_NW = _NC * _NS
_ROWS_PER_W = BATCH // _NW
_ROWS_PER_CHUNK = 4
_IDX_PER_CHUNK = _ROWS_PER_CHUNK * CTX
_CHUNKS = _ROWS_PER_W // _ROWS_PER_CHUNK


def _tree_sum16(vals):
    while len(vals) > 1:
        vals = [a + b for a, b in zip(vals[::2], vals[1::2])] + (
            [vals[-1]] if len(vals) % 2 else [])
    return vals[0]


def _emb_sum_body(x_hbm, table_hbm, out_hbm, idx_v, rows_v, acc_v, sem):
    wid = lax.axis_index("s") * _NC + lax.axis_index("c")
    base_row = wid * _ROWS_PER_W
    pltpu.sync_copy(x_hbm.at[pl.ds(base_row * CTX, _ROWS_PER_W * CTX)], idx_v)

    def gather(c, k):
        idx_slice = idx_v.at[pl.ds(c * _IDX_PER_CHUNK, _IDX_PER_CHUNK)]
        dst = rows_v.at[pl.ds(k * _IDX_PER_CHUNK, _IDX_PER_CHUNK)]
        return pltpu.make_async_copy(table_hbm.at[idx_slice], dst, sem)

    def super_body(s, carry):
        c0 = s * _SUPER
        for k in range(_SUPER):
            gather(c0 + k, k).start()
        for k in range(_SUPER):
            gather(c0 + k, k).wait()
        for rk in range(_SUPER * _ROWS_PER_CHUNK):
            row = c0 * _ROWS_PER_CHUNK + rk
            for g in range(EMBED_DIM // 16):
                vals = [rows_v[rk * CTX + j, pl.ds(g * 16, 16)]
                        for j in range(CTX)]
                acc_v[row, pl.ds(g * 16, 16)] = _tree_sum16(vals)
        return carry

    lax.fori_loop(0, _CHUNKS // _SUPER, super_body, 0)
    pltpu.sync_copy(acc_v, out_hbm.at[pl.ds(base_row, _ROWS_PER_W)])


_SUPER = 4


_emb_sum = functools.partial(
    pl.kernel,
    out_type=jax.ShapeDtypeStruct((BATCH, EMBED_DIM), jnp.float32),
    mesh=plsc.VectorSubcoreMesh(core_axis_name="c", subcore_axis_name="s"),
    scratch_types=[
        pltpu.VMEM((_ROWS_PER_W * CTX,), jnp.int32),
        pltpu.VMEM((_SUPER * _IDX_PER_CHUNK, EMBED_DIM), jnp.float32),
        pltpu.VMEM((_ROWS_PER_W, EMBED_DIM), jnp.float32),
        pltpu.SemaphoreType.DMA,
    ],
)(_emb_sum_body)


_V_BLK = 1024
_NV = pl.cdiv(VOCAB, _V_BLK)


def _proj_body(emb_ref, wt_ref, b_ref, out_ref, ebf_ref):
    @pl.when(pl.program_id(0) == 0)
    def _cast_once():
        ebf_ref[...] = emb_ref[...].astype(jnp.bfloat16)

    acc = lax.dot_general(wt_ref[...], ebf_ref[...], (((1,), (1,)), ((), ())),
                          preferred_element_type=jnp.float32)
    bt = jnp.transpose(b_ref[...], (1, 0))
    out_ref[...] = acc + bt


def _projection(embedded, WT, b2d):
    return pl.pallas_call(
        _proj_body,
        grid=(_NV,),
        in_specs=[
            pl.BlockSpec((BATCH, EMBED_DIM), lambda v: (0, 0)),
            pl.BlockSpec((_V_BLK, EMBED_DIM), lambda v: (v, 0)),
            pl.BlockSpec((1, _V_BLK), lambda v: (0, v)),
        ],
        out_specs=pl.BlockSpec((_V_BLK, BATCH), lambda v: (v, 0)),
        out_shape=jax.ShapeDtypeStruct((VOCAB, BATCH), jnp.float32),
        scratch_shapes=[pltpu.VMEM((BATCH, EMBED_DIM), jnp.bfloat16)],
    )(embedded, WT, b2d)


def kernel(x, emb_table, W, b):
    x_flat = x.reshape(-1).astype(jnp.int32)
    wt_bf = W.T.astype(jnp.bfloat16)
    embedded = _emb_sum(x_flat, emb_table)
    out_t = _projection(embedded, wt_bf, b.reshape(1, VOCAB))
    return out_t.T


# speedup vs baseline: 1.0394x; 1.0394x over previous
import functools

import jax
import jax.numpy as jnp
from jax import lax
from jax.experimental import pallas as pl
from jax.experimental.pallas import tpu as pltpu
from jax.experimental.pallas import tpu_sc as plsc

VOCAB = 100000
EMBED_DIM = 128
BATCH = 4096
CTX = 20

_INFO = plsc.get_sparse_core_info()
_NC, _NS = _INFO.num_cores, _INFO.num_subcores
_NW = _NC * _NS
_ROWS_PER_W = BATCH // _NW
_ROWS_PER_CHUNK = 4
_IDX_PER_CHUNK = _ROWS_PER_CHUNK * CTX
_CHUNKS = _ROWS_PER_W // _ROWS_PER_CHUNK


def _tree_sum16(vals):
    while len(vals) > 1:
        vals = [a + b for a, b in zip(vals[::2], vals[1::2])] + (
            [vals[-1]] if len(vals) % 2 else [])
    return vals[0]


def _emb_sum_body(x_hbm, table_hbm, out_hbm, idx_v, rows_v, acc_v, sem):
    wid = lax.axis_index("s") * _NC + lax.axis_index("c")
    base_row = wid * _ROWS_PER_W
    pltpu.sync_copy(x_hbm.at[pl.ds(base_row * CTX, _ROWS_PER_W * CTX)], idx_v)

    def gather(c, k):
        idx_slice = idx_v.at[pl.ds(c * _IDX_PER_CHUNK, _IDX_PER_CHUNK)]
        dst = rows_v.at[pl.ds(k * _IDX_PER_CHUNK, _IDX_PER_CHUNK)]
        return pltpu.make_async_copy(table_hbm.at[idx_slice], dst, sem)

    def accumulate(c, k):
        for r in range(_ROWS_PER_CHUNK):
            row = c * _ROWS_PER_CHUNK + r
            for g in range(EMBED_DIM // 16):
                vals = [rows_v[k * _IDX_PER_CHUNK + r * CTX + j,
                               pl.ds(g * 16, 16)] for j in range(CTX)]
                acc_v[row, pl.ds(g * 16, 16)] = _tree_sum16(vals)

    gather(0, 0).start()

    def pair_body(i, carry):
        ca = 2 * i
        gather(ca, 0).wait()
        gather(ca + 1, 1).start()
        accumulate(ca, 0)
        gather(ca + 1, 1).wait()
        gather((ca + 2) % _CHUNKS, 0).start()
        accumulate(ca + 1, 1)
        return carry

    lax.fori_loop(0, _CHUNKS // 2, pair_body, 0)
    gather(0, 0).wait()
    pltpu.sync_copy(acc_v, out_hbm.at[pl.ds(base_row, _ROWS_PER_W)])


_emb_sum = functools.partial(
    pl.kernel,
    out_type=jax.ShapeDtypeStruct((BATCH, EMBED_DIM), jnp.float32),
    mesh=plsc.VectorSubcoreMesh(core_axis_name="c", subcore_axis_name="s"),
    scratch_types=[
        pltpu.VMEM((_ROWS_PER_W * CTX,), jnp.int32),
        pltpu.VMEM((2 * _IDX_PER_CHUNK, EMBED_DIM), jnp.float32),
        pltpu.VMEM((_ROWS_PER_W, EMBED_DIM), jnp.float32),
        pltpu.SemaphoreType.DMA,
    ],
)(_emb_sum_body)


_V_BLK = 1024
_NV = pl.cdiv(VOCAB, _V_BLK)


def _proj_body(emb_ref, wt_ref, b_ref, out_ref, ebf_ref):
    @pl.when(pl.program_id(0) == 0)
    def _cast_once():
        ebf_ref[...] = emb_ref[...].astype(jnp.bfloat16)

    acc = lax.dot_general(wt_ref[...], ebf_ref[...], (((1,), (1,)), ((), ())),
                          preferred_element_type=jnp.float32)
    bt = jnp.transpose(b_ref[...], (1, 0))
    out_ref[...] = acc + bt


def _projection(embedded, WT, b2d):
    return pl.pallas_call(
        _proj_body,
        grid=(_NV,),
        in_specs=[
            pl.BlockSpec((BATCH, EMBED_DIM), lambda v: (0, 0)),
            pl.BlockSpec((_V_BLK, EMBED_DIM), lambda v: (v, 0)),
            pl.BlockSpec((1, _V_BLK), lambda v: (0, v)),
        ],
        out_specs=pl.BlockSpec((_V_BLK, BATCH), lambda v: (v, 0)),
        out_shape=jax.ShapeDtypeStruct((VOCAB, BATCH), jnp.float32),
        scratch_shapes=[pltpu.VMEM((BATCH, EMBED_DIM), jnp.bfloat16)],
    )(embedded, WT, b2d)


def kernel(x, emb_table, W, b):
    x_flat = x.reshape(-1).astype(jnp.int32)
    wt_bf = W.T.astype(jnp.bfloat16)
    embedded = _emb_sum(x_flat, emb_table)
    out_t = _projection(embedded, wt_bf, b.reshape(1, VOCAB))
    return out_t.T
